# baseline (speedup 1.0000x reference)
import functools

import jax
import jax.numpy as jnp
from jax import lax
from jax.experimental import pallas as pl
from jax.experimental.pallas import tpu as pltpu
from jax.experimental.pallas import tpu_sc as plsc

_MAX_REL = 20
_N = 1024
_D = 64
_V = 2 * _MAX_REL + 1
_LANES = 16

_NC, _NS = 2, 16
_NW = _NC * _NS
_ROWS_PER_W = _N // _NW

_TCOLS = 1280
_BCOLS = 48
_DT = _D // 8
_JT = _N // 128


@functools.partial(
    pl.kernel,
    out_type=jax.ShapeDtypeStruct((_N, _DT, _JT, 8, 128), jnp.float32),
    mesh=plsc.VectorSubcoreMesh(core_axis_name="c", subcore_axis_name="s"),
    scratch_types=[
        pltpu.VMEM((_D, _TCOLS), jnp.float32),
        pltpu.VMEM((_D, _BCOLS), jnp.float32),
        pltpu.SemaphoreType.DMA,
    ],
    compiler_params=pltpu.CompilerParams(
        use_tc_tiling_on_sc=False, needs_layout_passes=False
    ),
)
def _rel_pos_sc(bands_hbm, out_hbm, stript, tband, sem):
    wid = lax.axis_index("s") * _NC + lax.axis_index("c")
    a = lax.rem(wid, 8)
    b = lax.div(wid, 8)
    t_b = 228 + 256 * b + a
    s = lax.rem(a + 4, 8)
    t_w = t_b - s

    pltpu.sync_copy(bands_hbm.at[s], tband)

    iotas = [
        lax.iota(jnp.int32, _LANES) + c * _LANES
        for c in range(_TCOLS // _LANES)
    ]
    thresh = t_b + _MAX_REL

    def _build_row(d, carry):
        dsplat = jnp.full((_LANES,), d, dtype=jnp.int32)
        splat0 = plsc.load_gather(
            tband, [dsplat, jnp.full((_LANES,), s, dtype=jnp.int32)]
        )
        splat40 = plsc.load_gather(
            tband, [dsplat, jnp.full((_LANES,), s + _V - 1, dtype=jnp.int32)]
        )
        for c in range(_TCOLS // _LANES):
            stript[d, pl.ds(c * _LANES, _LANES)] = jnp.where(
                iotas[c] < thresh, splat0, splat40
            )
        for c in range(3):
            stript[d, pl.ds(t_w + c * _LANES, _LANES)] = tband[
                d, pl.ds(c * _LANES, _LANES)
            ]
        return carry

    lax.fori_loop(0, _D, _build_row, 0)

    def _issue_row(m):
        i = a + 256 * b + 8 * m
        t0 = 248 - 8 * m
        for dt in range(_DT):
            for jt in range(_JT):
                pltpu.async_copy(
                    stript.at[pl.ds(dt * 8, 8), pl.ds(t0 + jt * 128, 128)],
                    out_hbm.at[i, dt, jt],
                    sem,
                )

    def _drain_row():
        for _ in range(_DT * _JT):
            pltpu.make_async_copy(
                out_hbm.at[0, 0, 0],
                stript.at[pl.ds(0, 8), pl.ds(0, 128)],
                sem,
            ).wait()

    _issue_row(0)

    def _emit_row(m, carry):
        _issue_row(m)
        _drain_row()
        return carry

    lax.fori_loop(1, _ROWS_PER_W, _emit_row, 0)
    _drain_row()


@jax.jit
def _run(table):
    tablet = table.T
    col0 = tablet[:, :1]
    col40 = tablet[:, _V - 1 :]
    bands = jnp.stack(
        [
            jnp.concatenate(
                [
                    jnp.broadcast_to(col0, (_D, sh)),
                    tablet,
                    jnp.broadcast_to(col40, (_D, _BCOLS - _V - sh)),
                ],
                axis=1,
            )
            for sh in range(8)
        ]
    )
    phys = _rel_pos_sc(bands)
    return phys.transpose(0, 2, 4, 1, 3).reshape(_N, _N, _D)


def kernel(seq_len, table):
    del seq_len
    return _run(table)


# speedup vs baseline: 1.0439x; 1.0439x over previous
import functools

import jax
import jax.numpy as jnp
from jax import lax
from jax.experimental import pallas as pl
from jax.experimental.pallas import tpu as pltpu
from jax.experimental.pallas import tpu_sc as plsc

_MAX_REL = 20
_N = 1024
_D = 64
_V = 2 * _MAX_REL + 1
_LANES = 16

_NC, _NS = 2, 16
_NW = _NC * _NS
_ROWS_PER_W = _N // _NW

_TCOLS = 1280
_BCOLS = 48
_DT = _D // 8
_JT = _N // 128


@functools.partial(
    pl.kernel,
    out_type=jax.ShapeDtypeStruct((_N, _DT, _JT, 8, 128), jnp.float32),
    mesh=plsc.VectorSubcoreMesh(core_axis_name="c", subcore_axis_name="s"),
    scratch_types=[
        pltpu.VMEM((_D, _TCOLS), jnp.float32),
        pltpu.VMEM((_D, _BCOLS), jnp.float32),
        pltpu.SemaphoreType.DMA,
    ],
    compiler_params=pltpu.CompilerParams(
        use_tc_tiling_on_sc=False, needs_layout_passes=False
    ),
)
def _rel_pos_sc(bands_hbm, out_hbm, stript, tband, sem):
    wid = lax.axis_index("s") * _NC + lax.axis_index("c")
    a = lax.rem(wid, 8)
    b = lax.div(wid, 8)
    t_b = 228 + 256 * b + a
    s = lax.rem(a + 4, 8)
    t_w = t_b - s

    pltpu.sync_copy(bands_hbm.at[s], tband)

    iotas = [
        lax.iota(jnp.int32, _LANES) + c * _LANES
        for c in range(_TCOLS // _LANES)
    ]
    thresh = t_b + _MAX_REL

    def _build_row(d, carry):
        dsplat = jnp.full((_LANES,), d, dtype=jnp.int32)
        splat0 = plsc.load_gather(
            tband, [dsplat, jnp.full((_LANES,), s, dtype=jnp.int32)]
        )
        splat40 = plsc.load_gather(
            tband, [dsplat, jnp.full((_LANES,), s + _V - 1, dtype=jnp.int32)]
        )
        for c in range(_TCOLS // _LANES):
            stript[d, pl.ds(c * _LANES, _LANES)] = jnp.where(
                iotas[c] < thresh, splat0, splat40
            )
        for c in range(3):
            stript[d, pl.ds(t_w + c * _LANES, _LANES)] = tband[
                d, pl.ds(c * _LANES, _LANES)
            ]
        return carry

    lax.fori_loop(0, _D, _build_row, 0)

    def _emit_row(m, carry):
        i = a + 256 * b + 8 * m
        t0 = 248 - 8 * m
        copies = []
        for dt in range(_DT):
            for jt in range(_JT):
                copies.append(
                    pltpu.async_copy(
                        stript.at[
                            pl.ds(dt * 8, 8), pl.ds(t0 + jt * 128, 128)
                        ],
                        out_hbm.at[i, dt, jt],
                        sem,
                    )
                )
        for cp in copies:
            cp.wait()
        return carry

    lax.fori_loop(0, _ROWS_PER_W, _emit_row, 0)


@jax.jit
def _run(table):
    tablet = table.T
    col0 = tablet[:, :1]
    col40 = tablet[:, _V - 1 :]
    bands = jnp.stack(
        [
            jnp.concatenate(
                [
                    jnp.broadcast_to(col0, (_D, sh)),
                    tablet,
                    jnp.broadcast_to(col40, (_D, _BCOLS - _V - sh)),
                ],
                axis=1,
            )
            for sh in range(8)
        ]
    )
    phys = _rel_pos_sc(bands)
    return phys.transpose(0, 2, 4, 1, 3).reshape(_N, _N, _D)


def kernel(seq_len, table):
    del seq_len
    return _run(table)


# speedup vs baseline: 1.0465x; 1.0025x over previous
import functools

import jax
import jax.numpy as jnp
from jax import lax
from jax.experimental import pallas as pl
from jax.experimental.pallas import tpu as pltpu
from jax.experimental.pallas import tpu_sc as plsc

_MAX_REL = 20
_N = 1024
_D = 64
_V = 2 * _MAX_REL + 1
_LANES = 16

_NC, _NS = 2, 16
_NW = _NC * _NS
_ROWS_PER_W = _N // _NW

_TCOLS = 1280
_BCOLS = 48
_DT = _D // 8
_JT = _N // 128


@functools.partial(
    pl.kernel,
    out_type=jax.ShapeDtypeStruct((_N, _DT, _JT, 8, 128), jnp.float32),
    mesh=plsc.VectorSubcoreMesh(core_axis_name="c", subcore_axis_name="s"),
    scratch_types=[
        pltpu.VMEM((_D, _TCOLS), jnp.float32),
        pltpu.VMEM((_D, _BCOLS), jnp.float32),
        pltpu.VMEM((_JT, 8, 128), jnp.float32),
        pltpu.SemaphoreType.DMA,
    ],
    compiler_params=pltpu.CompilerParams(
        use_tc_tiling_on_sc=False, needs_layout_passes=False
    ),
)
def _rel_pos_sc(bands_hbm, out_hbm, stript, tband, drain8, sem):
    wid = lax.axis_index("s") * _NC + lax.axis_index("c")
    a = lax.rem(wid, 8)
    b = lax.div(wid, 8)
    t_b = 228 + 256 * b + a
    s = lax.rem(a + 4, 8)
    t_w = t_b - s

    pltpu.sync_copy(bands_hbm.at[s], tband)

    iotas = [
        lax.iota(jnp.int32, _LANES) + c * _LANES
        for c in range(_TCOLS // _LANES)
    ]
    thresh = t_b + _MAX_REL

    def _build_row(d, carry):
        dsplat = jnp.full((_LANES,), d, dtype=jnp.int32)
        splat0 = plsc.load_gather(
            tband, [dsplat, jnp.full((_LANES,), s, dtype=jnp.int32)]
        )
        splat40 = plsc.load_gather(
            tband, [dsplat, jnp.full((_LANES,), s + _V - 1, dtype=jnp.int32)]
        )
        for c in range(_TCOLS // _LANES):
            stript[d, pl.ds(c * _LANES, _LANES)] = jnp.where(
                iotas[c] < thresh, splat0, splat40
            )
        for c in range(3):
            stript[d, pl.ds(t_w + c * _LANES, _LANES)] = tband[
                d, pl.ds(c * _LANES, _LANES)
            ]
        return carry

    lax.fori_loop(0, _D, _build_row, 0)

    def _emit_row(m, carry):
        i = a + 256 * b + 8 * m
        t0 = 248 - 8 * m
        for dt in range(_DT):
            for jt in range(_JT):
                pltpu.async_copy(
                    stript.at[pl.ds(dt * 8, 8), pl.ds(t0 + jt * 128, 128)],
                    out_hbm.at[i, dt, jt],
                    sem,
                )
        for _ in range(_DT):
            pltpu.make_async_copy(out_hbm.at[0, 0], drain8, sem).wait()
        return carry

    lax.fori_loop(0, _ROWS_PER_W, _emit_row, 0)


@jax.jit
def _run(table):
    tablet = table.T
    col0 = tablet[:, :1]
    col40 = tablet[:, _V - 1 :]
    bands = jnp.stack(
        [
            jnp.concatenate(
                [
                    jnp.broadcast_to(col0, (_D, sh)),
                    tablet,
                    jnp.broadcast_to(col40, (_D, _BCOLS - _V - sh)),
                ],
                axis=1,
            )
            for sh in range(8)
        ]
    )
    phys = _rel_pos_sc(bands)
    return phys.transpose(0, 2, 4, 1, 3).reshape(_N, _N, _D)


def kernel(seq_len, table):
    del seq_len
    return _run(table)
